# Initial kernel scaffold; baseline (speedup 1.0000x reference)
import jax
import jax.numpy as jnp
from jax.experimental import pallas as pl


def kernel(x, edge_index, batch, W1_rel, b1, W1_root, g1, be1, W2_rel, b2, W2_root, g2, be2, W3_rel, b3, W3_root, Wl, bl):
    raise NotImplementedError("write your pallas kernel here")


# speedup vs baseline: 4.8632x; 4.8632x over previous
import functools

import jax
import jax.numpy as jnp
from jax import lax
from jax.experimental import pallas as pl
from jax.experimental.pallas import tpu as pltpu
from jax.experimental.pallas import tpu_sc as plsc

N = 10000
E = 160000
G = 64
C = 36

NC = 2
NS = 16
K = 128
CH = 40
E_PAD = NC * NS * CH * K
NPAD = 10240
ZCH = NPAD // NS // K
STRIPE = 624
TAIL = N - NS * STRIPE
OCH = [K, K, K, K, STRIPE - 4 * K]

BR = 2000


def _sc_segment_sum(F):
  mesh = plsc.VectorSubcoreMesh(core_axis_name="c", subcore_axis_name="s",
                                num_cores=NC, num_subcores=NS)

  @functools.partial(
      pl.kernel,
      out_type=jax.ShapeDtypeStruct((2 * N, F), jnp.float32),
      mesh=mesh,
      scratch_types=[
          pltpu.VMEM((CH, K), jnp.int32),
          pltpu.VMEM((CH, K), jnp.int32),
          pltpu.VMEM((K, F), jnp.float32),
          pltpu.VMEM_SHARED((NPAD, F), jnp.float32),
          pltpu.SemaphoreType.DMA,
      ],
      compiler_params=pltpu.CompilerParams(use_tc_tiling_on_sc=False),
  )
  def sc(y_hbm, srcm_hbm, dstm_hbm, zer_hbm, out_hbm,
         src_v, dst_v, rows_v, acc_sh, sem):
    cid = lax.axis_index("c")
    sid = lax.axis_index("s")
    wid = cid * NS + sid
    pltpu.sync_copy(srcm_hbm.at[wid], src_v)
    pltpu.sync_copy(dstm_hbm.at[wid], dst_v)
    pltpu.sync_copy(zer_hbm, rows_v)
    for z in range(ZCH):
      pltpu.sync_copy(rows_v, acc_sh.at[pl.ds(sid * (ZCH * K) + z * K, K)])
    plsc.subcore_barrier()

    def body(c, carry):
      pltpu.async_copy(y_hbm.at[src_v.at[c]], rows_v, sem).wait()
      pltpu.sync_copy(rows_v, acc_sh.at[dst_v.at[c]], add=True)
      return carry

    lax.fori_loop(0, CH, body, 0)
    plsc.subcore_barrier()
    off = 0
    for w in OCH:
      r0 = sid * STRIPE + off
      pltpu.sync_copy(acc_sh.at[pl.ds(r0, w)], rows_v.at[pl.ds(0, w)])
      pltpu.sync_copy(rows_v.at[pl.ds(0, w)],
                      out_hbm.at[pl.ds(cid * N + r0, w)])
      off += w

    @pl.when(sid == NS - 1)
    def _():
      t0 = NS * STRIPE
      pltpu.sync_copy(acc_sh.at[pl.ds(t0, TAIL)], rows_v.at[pl.ds(0, TAIL)])
      pltpu.sync_copy(rows_v.at[pl.ds(0, TAIL)],
                      out_hbm.at[pl.ds(cid * N + t0, TAIL)])

  return sc


def _mm_body(x_ref, wr_ref, wt_ref, y_ref, r_ref):
  xb = x_ref[...]
  dn = (((1,), (1,)), ((), ()))
  y_ref[...] = lax.dot_general(xb, wr_ref[...], dn,
                               preferred_element_type=jnp.float32)
  r_ref[...] = lax.dot_general(xb, wt_ref[...], dn,
                               preferred_element_type=jnp.float32)


def _mm(x, w_rel, w_root):
  fin = x.shape[1]
  fout = w_rel.shape[0]
  grid = N // BR
  return pl.pallas_call(
      _mm_body,
      grid=(grid,),
      in_specs=[
          pl.BlockSpec((BR, fin), lambda i: (i, 0)),
          pl.BlockSpec((fout, fin), lambda i: (0, 0)),
          pl.BlockSpec((fout, fin), lambda i: (0, 0)),
      ],
      out_specs=[
          pl.BlockSpec((BR, fout), lambda i: (i, 0)),
          pl.BlockSpec((BR, fout), lambda i: (i, 0)),
      ],
      out_shape=[
          jax.ShapeDtypeStruct((N, fout), jnp.float32),
          jax.ShapeDtypeStruct((N, fout), jnp.float32),
      ],
  )(x, w_rel, w_root)


def _stats_body(p0_ref, p1_ref, r_ref, z_ref, s_ref):
  i = pl.program_id(0)
  z = p0_ref[...] + p1_ref[...] + r_ref[...]
  z_ref[...] = z
  cs = jnp.sum(z, axis=0, keepdims=True)
  cq = jnp.sum(z * z, axis=0, keepdims=True)
  blk = jnp.concatenate(
      [cs, cq, jnp.zeros((6, z.shape[1]), jnp.float32)], axis=0)

  @pl.when(i == 0)
  def _():
    s_ref[...] = blk

  @pl.when(i != 0)
  def _():
    s_ref[...] = s_ref[...] + blk


def _stats(p, r):
  f = r.shape[1]
  grid = N // BR
  return pl.pallas_call(
      _stats_body,
      grid=(grid,),
      in_specs=[
          pl.BlockSpec((BR, f), lambda i: (i, 0)),
          pl.BlockSpec((BR, f), lambda i: (i + N // BR, 0)),
          pl.BlockSpec((BR, f), lambda i: (i, 0)),
      ],
      out_specs=[
          pl.BlockSpec((BR, f), lambda i: (i, 0)),
          pl.BlockSpec((8, f), lambda i: (0, 0)),
      ],
      out_shape=[
          jax.ShapeDtypeStruct((N, f), jnp.float32),
          jax.ShapeDtypeStruct((8, f), jnp.float32),
      ],
  )(p, p, r)


def _bnmm_body(z_ref, s_ref, g_ref, be_ref, wr_ref, wt_ref, y_ref, r_ref):
  s = s_ref[...]
  mu = s[0:1, :] * (1.0 / N)
  var = s[1:2, :] * (1.0 / N) - mu * mu
  scale = g_ref[...] * lax.rsqrt(var + 1e-5)
  shift = be_ref[...] - mu * scale
  h = jnp.maximum(z_ref[...] * scale + shift, 0.0)
  dn = (((1,), (1,)), ((), ()))
  y_ref[...] = lax.dot_general(h, wr_ref[...], dn,
                               preferred_element_type=jnp.float32)
  r_ref[...] = lax.dot_general(h, wt_ref[...], dn,
                               preferred_element_type=jnp.float32)


def _bnmm(z, s, g, be, w_rel, w_root):
  fin = z.shape[1]
  fout = w_rel.shape[0]
  grid = N // BR
  return pl.pallas_call(
      _bnmm_body,
      grid=(grid,),
      in_specs=[
          pl.BlockSpec((BR, fin), lambda i: (i, 0)),
          pl.BlockSpec((8, fin), lambda i: (0, 0)),
          pl.BlockSpec((1, fin), lambda i: (0, 0)),
          pl.BlockSpec((1, fin), lambda i: (0, 0)),
          pl.BlockSpec((fout, fin), lambda i: (0, 0)),
          pl.BlockSpec((fout, fin), lambda i: (0, 0)),
      ],
      out_specs=[
          pl.BlockSpec((BR, fout), lambda i: (i, 0)),
          pl.BlockSpec((BR, fout), lambda i: (i, 0)),
      ],
      out_shape=[
          jax.ShapeDtypeStruct((N, fout), jnp.float32),
          jax.ShapeDtypeStruct((N, fout), jnp.float32),
      ],
  )(z, s, g.reshape(1, fin), be.reshape(1, fin), w_rel, w_root)


def _final_body(p0_ref, p1_ref, r_ref, b3_ref, bt_ref, wl_ref, bl_ref,
                out_ref, acc_ref):
  i = pl.program_id(0)

  @pl.when(i == 0)
  def _():
    acc_ref[...] = jnp.zeros_like(acc_ref)

  h = jnp.maximum(p0_ref[...] + p1_ref[...] + r_ref[...] + b3_ref[...], 0.0)
  bt = bt_ref[0, 0, :]
  oh = (lax.broadcasted_iota(jnp.int32, (G, BR), 0) == bt[None, :]).astype(
      jnp.float32)
  hcat = jnp.concatenate(
      [h, jnp.ones((BR, 1), jnp.float32), jnp.zeros((BR, 31), jnp.float32)],
      axis=1)
  acc_ref[...] += lax.dot_general(oh, hcat, (((1,), (0,)), ((), ())),
                                  preferred_element_type=jnp.float32)

  @pl.when(i == pl.num_programs(0) - 1)
  def _():
    acc = acc_ref[...]
    pooled = acc[:, :32] / jnp.maximum(acc[:, 32:33], 1.0)
    logits = lax.dot_general(pooled, wl_ref[...], (((1,), (1,)), ((), ())),
                             preferred_element_type=jnp.float32)
    logits = logits + bl_ref[...]
    m = jnp.max(logits, axis=1, keepdims=True)
    e = jnp.exp(logits - m)
    out_ref[...] = logits - m - jnp.log(jnp.sum(e, axis=1, keepdims=True))


def _final(p, r3, b3, batch3, wl, bl):
  grid = N // BR
  return pl.pallas_call(
      _final_body,
      grid=(grid,),
      in_specs=[
          pl.BlockSpec((BR, 32), lambda i: (i, 0)),
          pl.BlockSpec((BR, 32), lambda i: (i + N // BR, 0)),
          pl.BlockSpec((BR, 32), lambda i: (i, 0)),
          pl.BlockSpec((1, 32), lambda i: (0, 0)),
          pl.BlockSpec((1, 1, BR), lambda i: (i, 0, 0)),
          pl.BlockSpec((C, 32), lambda i: (0, 0)),
          pl.BlockSpec((1, C), lambda i: (0, 0)),
      ],
      out_specs=pl.BlockSpec((G, C), lambda i: (0, 0)),
      out_shape=jax.ShapeDtypeStruct((G, C), jnp.float32),
      scratch_shapes=[pltpu.VMEM((G, 64), jnp.float32)],
  )(p, p, r3, b3.reshape(1, 32), batch3, wl, bl.reshape(1, C))


def kernel(x, edge_index, batch, W1_rel, b1, W1_root, g1, be1, W2_rel, b2,
           W2_root, g2, be2, W3_rel, b3, W3_root, Wl, bl):
  src = edge_index[0]
  dst = edge_index[1]
  pad = E_PAD - E
  srcm = jnp.concatenate([src, jnp.zeros((pad,), jnp.int32)]).reshape(
      NC * NS, CH, K)
  dstm = jnp.concatenate([dst, jnp.full((pad,), N, jnp.int32)]).reshape(
      NC * NS, CH, K)

  sc128 = _sc_segment_sum(128)
  sc64 = _sc_segment_sum(64)
  sc32 = _sc_segment_sum(32)
  z128 = jnp.zeros((K, 128), jnp.float32)
  z64 = jnp.zeros((K, 64), jnp.float32)
  z32 = jnp.zeros((K, 32), jnp.float32)

  y1, r1 = _mm(x, W1_rel, W1_root)
  p1 = sc128(y1, srcm, dstm, z128)
  zz1, s1 = _stats(p1, r1)
  y2, r2 = _bnmm(zz1, s1, g1, be1, W2_rel, W2_root)
  p2 = sc64(y2, srcm, dstm, z64)
  zz2, s2 = _stats(p2, r2)
  y3, r3 = _bnmm(zz2, s2, g2, be2, W3_rel, W3_root)
  p3 = sc32(y3, srcm, dstm, z32)
  batch3 = batch.reshape(N // BR, 1, BR)
  return _final(p3, r3, b3, batch3, Wl, bl)


# speedup vs baseline: 5.2605x; 1.0817x over previous
import functools

import jax
import jax.numpy as jnp
from jax import lax
from jax.experimental import pallas as pl
from jax.experimental.pallas import tpu as pltpu
from jax.experimental.pallas import tpu_sc as plsc

N = 10000
E = 160000
G = 64
C = 36

NC = 2
NS = 16
K = 128
CH = 40
E_PAD = NC * NS * CH * K
NPAD = 10240
ZCH = NPAD // NS // K
STRIPE = 624
TAIL = N - NS * STRIPE
OCH = [K, K, K, K, STRIPE - 4 * K]

BR = 2000


def _sc_segment_sum(F):
  mesh = plsc.VectorSubcoreMesh(core_axis_name="c", subcore_axis_name="s",
                                num_cores=NC, num_subcores=NS)

  @functools.partial(
      pl.kernel,
      out_type=jax.ShapeDtypeStruct((2 * N, F), jnp.float32),
      mesh=mesh,
      scratch_types=[
          pltpu.VMEM((CH, K), jnp.int32),
          pltpu.VMEM((CH, K), jnp.int32),
          pltpu.VMEM((K, F), jnp.float32),
          pltpu.VMEM((K, F), jnp.float32),
          pltpu.VMEM_SHARED((NPAD, F), jnp.float32),
          pltpu.SemaphoreType.DMA,
          pltpu.SemaphoreType.DMA,
      ],
      compiler_params=pltpu.CompilerParams(use_tc_tiling_on_sc=False),
  )
  def sc(y_hbm, srcm_hbm, dstm_hbm, zer_hbm, out_hbm,
         src_v, dst_v, rows0_v, rows1_v, acc_sh, sem0, sem1):
    cid = lax.axis_index("c")
    sid = lax.axis_index("s")
    wid = cid * NS + sid
    pltpu.sync_copy(srcm_hbm.at[wid], src_v)
    pltpu.sync_copy(dstm_hbm.at[wid], dst_v)
    pltpu.sync_copy(zer_hbm, rows0_v)
    for z in range(ZCH):
      pltpu.sync_copy(rows0_v, acc_sh.at[pl.ds(sid * (ZCH * K) + z * K, K)])
    plsc.subcore_barrier()

    pltpu.async_copy(y_hbm.at[src_v.at[0]], rows0_v, sem0)

    def body(c2, carry):
      c = 2 * c2
      pltpu.make_async_copy(y_hbm.at[src_v.at[c]], rows0_v, sem0).wait()
      pltpu.async_copy(y_hbm.at[src_v.at[c + 1]], rows1_v, sem1)
      pltpu.sync_copy(rows0_v, acc_sh.at[dst_v.at[c]], add=True)
      pltpu.make_async_copy(y_hbm.at[src_v.at[c + 1]], rows1_v, sem1).wait()

      @pl.when(c + 2 < CH)
      def _():
        pltpu.async_copy(y_hbm.at[src_v.at[c + 2]], rows0_v, sem0)

      pltpu.sync_copy(rows1_v, acc_sh.at[dst_v.at[c + 1]], add=True)
      return carry

    lax.fori_loop(0, CH // 2, body, 0)
    plsc.subcore_barrier()
    off = 0
    for w in OCH:
      r0 = sid * STRIPE + off
      pltpu.sync_copy(acc_sh.at[pl.ds(r0, w)], rows0_v.at[pl.ds(0, w)])
      pltpu.sync_copy(rows0_v.at[pl.ds(0, w)],
                      out_hbm.at[pl.ds(cid * N + r0, w)])
      off += w

    @pl.when(sid == NS - 1)
    def _():
      t0 = NS * STRIPE
      pltpu.sync_copy(acc_sh.at[pl.ds(t0, TAIL)], rows0_v.at[pl.ds(0, TAIL)])
      pltpu.sync_copy(rows0_v.at[pl.ds(0, TAIL)],
                      out_hbm.at[pl.ds(cid * N + t0, TAIL)])

  return sc


def _mm_body(x_ref, wr_ref, wt_ref, y_ref, r_ref):
  xb = x_ref[...]
  dn = (((1,), (1,)), ((), ()))
  y_ref[...] = lax.dot_general(xb, wr_ref[...], dn,
                               preferred_element_type=jnp.float32)
  r_ref[...] = lax.dot_general(xb, wt_ref[...], dn,
                               preferred_element_type=jnp.float32)


def _mm(x, w_rel, w_root):
  fin = x.shape[1]
  fout = w_rel.shape[0]
  grid = N // BR
  return pl.pallas_call(
      _mm_body,
      grid=(grid,),
      in_specs=[
          pl.BlockSpec((BR, fin), lambda i: (i, 0)),
          pl.BlockSpec((fout, fin), lambda i: (0, 0)),
          pl.BlockSpec((fout, fin), lambda i: (0, 0)),
      ],
      out_specs=[
          pl.BlockSpec((BR, fout), lambda i: (i, 0)),
          pl.BlockSpec((BR, fout), lambda i: (i, 0)),
      ],
      out_shape=[
          jax.ShapeDtypeStruct((N, fout), jnp.float32),
          jax.ShapeDtypeStruct((N, fout), jnp.float32),
      ],
  )(x, w_rel, w_root)


def _stats_body(p0_ref, p1_ref, r_ref, z_ref, s_ref):
  i = pl.program_id(0)
  z = p0_ref[...] + p1_ref[...] + r_ref[...]
  z_ref[...] = z
  cs = jnp.sum(z, axis=0, keepdims=True)
  cq = jnp.sum(z * z, axis=0, keepdims=True)
  blk = jnp.concatenate(
      [cs, cq, jnp.zeros((6, z.shape[1]), jnp.float32)], axis=0)

  @pl.when(i == 0)
  def _():
    s_ref[...] = blk

  @pl.when(i != 0)
  def _():
    s_ref[...] = s_ref[...] + blk


def _stats(p, r):
  f = r.shape[1]
  grid = N // BR
  return pl.pallas_call(
      _stats_body,
      grid=(grid,),
      in_specs=[
          pl.BlockSpec((BR, f), lambda i: (i, 0)),
          pl.BlockSpec((BR, f), lambda i: (i + N // BR, 0)),
          pl.BlockSpec((BR, f), lambda i: (i, 0)),
      ],
      out_specs=[
          pl.BlockSpec((BR, f), lambda i: (i, 0)),
          pl.BlockSpec((8, f), lambda i: (0, 0)),
      ],
      out_shape=[
          jax.ShapeDtypeStruct((N, f), jnp.float32),
          jax.ShapeDtypeStruct((8, f), jnp.float32),
      ],
  )(p, p, r)


def _bnmm_body(z_ref, s_ref, g_ref, be_ref, wr_ref, wt_ref, y_ref, r_ref):
  s = s_ref[...]
  mu = s[0:1, :] * (1.0 / N)
  var = s[1:2, :] * (1.0 / N) - mu * mu
  scale = g_ref[...] * lax.rsqrt(var + 1e-5)
  shift = be_ref[...] - mu * scale
  h = jnp.maximum(z_ref[...] * scale + shift, 0.0)
  dn = (((1,), (1,)), ((), ()))
  y_ref[...] = lax.dot_general(h, wr_ref[...], dn,
                               preferred_element_type=jnp.float32)
  r_ref[...] = lax.dot_general(h, wt_ref[...], dn,
                               preferred_element_type=jnp.float32)


def _bnmm(z, s, g, be, w_rel, w_root):
  fin = z.shape[1]
  fout = w_rel.shape[0]
  grid = N // BR
  return pl.pallas_call(
      _bnmm_body,
      grid=(grid,),
      in_specs=[
          pl.BlockSpec((BR, fin), lambda i: (i, 0)),
          pl.BlockSpec((8, fin), lambda i: (0, 0)),
          pl.BlockSpec((1, fin), lambda i: (0, 0)),
          pl.BlockSpec((1, fin), lambda i: (0, 0)),
          pl.BlockSpec((fout, fin), lambda i: (0, 0)),
          pl.BlockSpec((fout, fin), lambda i: (0, 0)),
      ],
      out_specs=[
          pl.BlockSpec((BR, fout), lambda i: (i, 0)),
          pl.BlockSpec((BR, fout), lambda i: (i, 0)),
      ],
      out_shape=[
          jax.ShapeDtypeStruct((N, fout), jnp.float32),
          jax.ShapeDtypeStruct((N, fout), jnp.float32),
      ],
  )(z, s, g.reshape(1, fin), be.reshape(1, fin), w_rel, w_root)


def _final_body(p0_ref, p1_ref, r_ref, b3_ref, bt_ref, wl_ref, bl_ref,
                out_ref, acc_ref):
  i = pl.program_id(0)

  @pl.when(i == 0)
  def _():
    acc_ref[...] = jnp.zeros_like(acc_ref)

  h = jnp.maximum(p0_ref[...] + p1_ref[...] + r_ref[...] + b3_ref[...], 0.0)
  bt = bt_ref[0, 0, :]
  oh = (lax.broadcasted_iota(jnp.int32, (G, BR), 0) == bt[None, :]).astype(
      jnp.float32)
  hcat = jnp.concatenate(
      [h, jnp.ones((BR, 1), jnp.float32), jnp.zeros((BR, 31), jnp.float32)],
      axis=1)
  acc_ref[...] += lax.dot_general(oh, hcat, (((1,), (0,)), ((), ())),
                                  preferred_element_type=jnp.float32)

  @pl.when(i == pl.num_programs(0) - 1)
  def _():
    acc = acc_ref[...]
    pooled = acc[:, :32] / jnp.maximum(acc[:, 32:33], 1.0)
    logits = lax.dot_general(pooled, wl_ref[...], (((1,), (1,)), ((), ())),
                             preferred_element_type=jnp.float32)
    logits = logits + bl_ref[...]
    m = jnp.max(logits, axis=1, keepdims=True)
    e = jnp.exp(logits - m)
    out_ref[...] = logits - m - jnp.log(jnp.sum(e, axis=1, keepdims=True))


def _final(p, r3, b3, batch3, wl, bl):
  grid = N // BR
  return pl.pallas_call(
      _final_body,
      grid=(grid,),
      in_specs=[
          pl.BlockSpec((BR, 32), lambda i: (i, 0)),
          pl.BlockSpec((BR, 32), lambda i: (i + N // BR, 0)),
          pl.BlockSpec((BR, 32), lambda i: (i, 0)),
          pl.BlockSpec((1, 32), lambda i: (0, 0)),
          pl.BlockSpec((1, 1, BR), lambda i: (i, 0, 0)),
          pl.BlockSpec((C, 32), lambda i: (0, 0)),
          pl.BlockSpec((1, C), lambda i: (0, 0)),
      ],
      out_specs=pl.BlockSpec((G, C), lambda i: (0, 0)),
      out_shape=jax.ShapeDtypeStruct((G, C), jnp.float32),
      scratch_shapes=[pltpu.VMEM((G, 64), jnp.float32)],
  )(p, p, r3, b3.reshape(1, 32), batch3, wl, bl.reshape(1, C))


def kernel(x, edge_index, batch, W1_rel, b1, W1_root, g1, be1, W2_rel, b2,
           W2_root, g2, be2, W3_rel, b3, W3_root, Wl, bl):
  src = edge_index[0]
  dst = edge_index[1]
  pad = E_PAD - E
  srcm = jnp.concatenate([src, jnp.zeros((pad,), jnp.int32)]).reshape(
      NC * NS, CH, K)
  dstm = jnp.concatenate([dst, jnp.full((pad,), N, jnp.int32)]).reshape(
      NC * NS, CH, K)

  sc128 = _sc_segment_sum(128)
  sc64 = _sc_segment_sum(64)
  sc32 = _sc_segment_sum(32)
  z128 = jnp.zeros((K, 128), jnp.float32)
  z64 = jnp.zeros((K, 64), jnp.float32)
  z32 = jnp.zeros((K, 32), jnp.float32)

  y1, r1 = _mm(x, W1_rel, W1_root)
  p1 = sc128(y1, srcm, dstm, z128)
  zz1, s1 = _stats(p1, r1)
  y2, r2 = _bnmm(zz1, s1, g1, be1, W2_rel, W2_root)
  p2 = sc64(y2, srcm, dstm, z64)
  zz2, s2 = _stats(p2, r2)
  y3, r3 = _bnmm(zz2, s2, g2, be2, W3_rel, W3_root)
  p3 = sc32(y3, srcm, dstm, z32)
  batch3 = batch.reshape(N // BR, 1, BR)
  return _final(p3, r3, b3, batch3, Wl, bl)


# speedup vs baseline: 6.6640x; 1.2668x over previous
import functools

import jax
import jax.numpy as jnp
from jax import lax
from jax.experimental import pallas as pl
from jax.experimental.pallas import tpu as pltpu
from jax.experimental.pallas import tpu_sc as plsc

N = 10000
E = 160000
G = 64
C = 36

NC = 2
NS = 16
K = 128
CH = 40
E_PAD = NC * NS * CH * K
NPAD = 10240
ZCH = NPAD // NS // K
STRIPE = 624
TAIL = N - NS * STRIPE
OCH = [K, K, K, K, STRIPE - 4 * K]

BR = 2000


def _sc_segment_sum(F, stage_y=False):
  mesh = plsc.VectorSubcoreMesh(core_axis_name="c", subcore_axis_name="s",
                                num_cores=NC, num_subcores=NS)
  scratch = [
      pltpu.VMEM((CH, K), jnp.int32),
      pltpu.VMEM((CH, K), jnp.int32),
      pltpu.VMEM((K, F), jnp.float32),
      pltpu.VMEM((K, F), jnp.float32),
      pltpu.VMEM_SHARED((NPAD, F), jnp.float32),
      pltpu.SemaphoreType.DMA,
      pltpu.SemaphoreType.DMA,
  ]
  if stage_y:
    scratch.append(pltpu.VMEM_SHARED((N, F), jnp.float32))

  @functools.partial(
      pl.kernel,
      out_type=jax.ShapeDtypeStruct((2 * N, F), jnp.float32),
      mesh=mesh,
      scratch_types=scratch,
      compiler_params=pltpu.CompilerParams(use_tc_tiling_on_sc=False),
  )
  def sc(y_hbm, srcm_hbm, dstm_hbm, zer_hbm, out_hbm,
         src_v, dst_v, rows0_v, rows1_v, acc_sh, sem0, sem1, *maybe_ysh):
    cid = lax.axis_index("c")
    sid = lax.axis_index("s")
    wid = cid * NS + sid
    pltpu.sync_copy(srcm_hbm.at[wid], src_v)
    pltpu.sync_copy(dstm_hbm.at[wid], dst_v)
    if stage_y:
      ysh = maybe_ysh[0]
      off = 0
      for w in OCH:
        r0 = sid * STRIPE + off
        pltpu.sync_copy(y_hbm.at[pl.ds(r0, w)], rows1_v.at[pl.ds(0, w)])
        pltpu.sync_copy(rows1_v.at[pl.ds(0, w)], ysh.at[pl.ds(r0, w)])
        off += w

      @pl.when(sid == NS - 1)
      def _():
        t0 = NS * STRIPE
        pltpu.sync_copy(y_hbm.at[pl.ds(t0, TAIL)], rows1_v.at[pl.ds(0, TAIL)])
        pltpu.sync_copy(rows1_v.at[pl.ds(0, TAIL)], ysh.at[pl.ds(t0, TAIL)])

      ysrc = ysh
    else:
      ysrc = y_hbm
    pltpu.sync_copy(zer_hbm, rows0_v)
    for z in range(ZCH):
      pltpu.sync_copy(rows0_v, acc_sh.at[pl.ds(sid * (ZCH * K) + z * K, K)])
    plsc.subcore_barrier()

    pltpu.async_copy(ysrc.at[src_v.at[0]], rows0_v, sem0)

    def body(c2, carry):
      c = 2 * c2
      pltpu.make_async_copy(ysrc.at[src_v.at[c]], rows0_v, sem0).wait()
      pltpu.async_copy(ysrc.at[src_v.at[c + 1]], rows1_v, sem1)
      pltpu.sync_copy(rows0_v, acc_sh.at[dst_v.at[c]], add=True)
      pltpu.make_async_copy(ysrc.at[src_v.at[c + 1]], rows1_v, sem1).wait()

      @pl.when(c + 2 < CH)
      def _():
        pltpu.async_copy(ysrc.at[src_v.at[c + 2]], rows0_v, sem0)

      pltpu.sync_copy(rows1_v, acc_sh.at[dst_v.at[c + 1]], add=True)
      return carry

    lax.fori_loop(0, CH // 2, body, 0)
    plsc.subcore_barrier()
    off = 0
    for w in OCH:
      r0 = sid * STRIPE + off
      pltpu.sync_copy(acc_sh.at[pl.ds(r0, w)], rows0_v.at[pl.ds(0, w)])
      pltpu.sync_copy(rows0_v.at[pl.ds(0, w)],
                      out_hbm.at[pl.ds(cid * N + r0, w)])
      off += w

    @pl.when(sid == NS - 1)
    def _():
      t0 = NS * STRIPE
      pltpu.sync_copy(acc_sh.at[pl.ds(t0, TAIL)], rows0_v.at[pl.ds(0, TAIL)])
      pltpu.sync_copy(rows0_v.at[pl.ds(0, TAIL)],
                      out_hbm.at[pl.ds(cid * N + t0, TAIL)])

  return sc


def _mm_body(x_ref, wr_ref, wt_ref, y_ref, r_ref):
  xb = x_ref[...]
  dn = (((1,), (1,)), ((), ()))
  y_ref[...] = lax.dot_general(xb, wr_ref[...], dn,
                               preferred_element_type=jnp.float32)
  r_ref[...] = lax.dot_general(xb, wt_ref[...], dn,
                               preferred_element_type=jnp.float32)


def _mm(x, w_rel, w_root):
  fin = x.shape[1]
  fout = w_rel.shape[0]
  grid = N // BR
  return pl.pallas_call(
      _mm_body,
      grid=(grid,),
      in_specs=[
          pl.BlockSpec((BR, fin), lambda i: (i, 0)),
          pl.BlockSpec((fout, fin), lambda i: (0, 0)),
          pl.BlockSpec((fout, fin), lambda i: (0, 0)),
      ],
      out_specs=[
          pl.BlockSpec((BR, fout), lambda i: (i, 0)),
          pl.BlockSpec((BR, fout), lambda i: (i, 0)),
      ],
      out_shape=[
          jax.ShapeDtypeStruct((N, fout), jnp.float32),
          jax.ShapeDtypeStruct((N, fout), jnp.float32),
      ],
  )(x, w_rel, w_root)


def _stats_body(p0_ref, p1_ref, r_ref, z_ref, s_ref):
  i = pl.program_id(0)
  z = p0_ref[...] + p1_ref[...] + r_ref[...]
  z_ref[...] = z
  cs = jnp.sum(z, axis=0, keepdims=True)
  cq = jnp.sum(z * z, axis=0, keepdims=True)
  blk = jnp.concatenate(
      [cs, cq, jnp.zeros((6, z.shape[1]), jnp.float32)], axis=0)

  @pl.when(i == 0)
  def _():
    s_ref[...] = blk

  @pl.when(i != 0)
  def _():
    s_ref[...] = s_ref[...] + blk


def _stats(p, r):
  f = r.shape[1]
  grid = N // BR
  return pl.pallas_call(
      _stats_body,
      grid=(grid,),
      in_specs=[
          pl.BlockSpec((BR, f), lambda i: (i, 0)),
          pl.BlockSpec((BR, f), lambda i: (i + N // BR, 0)),
          pl.BlockSpec((BR, f), lambda i: (i, 0)),
      ],
      out_specs=[
          pl.BlockSpec((BR, f), lambda i: (i, 0)),
          pl.BlockSpec((8, f), lambda i: (0, 0)),
      ],
      out_shape=[
          jax.ShapeDtypeStruct((N, f), jnp.float32),
          jax.ShapeDtypeStruct((8, f), jnp.float32),
      ],
  )(p, p, r)


def _bnmm_body(z_ref, s_ref, g_ref, be_ref, wr_ref, wt_ref, y_ref, r_ref):
  s = s_ref[...]
  mu = s[0:1, :] * (1.0 / N)
  var = s[1:2, :] * (1.0 / N) - mu * mu
  scale = g_ref[...] * lax.rsqrt(var + 1e-5)
  shift = be_ref[...] - mu * scale
  h = jnp.maximum(z_ref[...] * scale + shift, 0.0)
  dn = (((1,), (1,)), ((), ()))
  y_ref[...] = lax.dot_general(h, wr_ref[...], dn,
                               preferred_element_type=jnp.float32)
  r_ref[...] = lax.dot_general(h, wt_ref[...], dn,
                               preferred_element_type=jnp.float32)


def _bnmm(z, s, g, be, w_rel, w_root):
  fin = z.shape[1]
  fout = w_rel.shape[0]
  grid = N // BR
  return pl.pallas_call(
      _bnmm_body,
      grid=(grid,),
      in_specs=[
          pl.BlockSpec((BR, fin), lambda i: (i, 0)),
          pl.BlockSpec((8, fin), lambda i: (0, 0)),
          pl.BlockSpec((1, fin), lambda i: (0, 0)),
          pl.BlockSpec((1, fin), lambda i: (0, 0)),
          pl.BlockSpec((fout, fin), lambda i: (0, 0)),
          pl.BlockSpec((fout, fin), lambda i: (0, 0)),
      ],
      out_specs=[
          pl.BlockSpec((BR, fout), lambda i: (i, 0)),
          pl.BlockSpec((BR, fout), lambda i: (i, 0)),
      ],
      out_shape=[
          jax.ShapeDtypeStruct((N, fout), jnp.float32),
          jax.ShapeDtypeStruct((N, fout), jnp.float32),
      ],
  )(z, s, g.reshape(1, fin), be.reshape(1, fin), w_rel, w_root)


def _final_body(p0_ref, p1_ref, r_ref, b3_ref, bt_ref, wl_ref, bl_ref,
                out_ref, acc_ref):
  i = pl.program_id(0)

  @pl.when(i == 0)
  def _():
    acc_ref[...] = jnp.zeros_like(acc_ref)

  h = jnp.maximum(p0_ref[...] + p1_ref[...] + r_ref[...] + b3_ref[...], 0.0)
  bt = bt_ref[0, 0, :]
  oh = (lax.broadcasted_iota(jnp.int32, (G, BR), 0) == bt[None, :]).astype(
      jnp.float32)
  hcat = jnp.concatenate(
      [h, jnp.ones((BR, 1), jnp.float32), jnp.zeros((BR, 31), jnp.float32)],
      axis=1)
  acc_ref[...] += lax.dot_general(oh, hcat, (((1,), (0,)), ((), ())),
                                  preferred_element_type=jnp.float32)

  @pl.when(i == pl.num_programs(0) - 1)
  def _():
    acc = acc_ref[...]
    pooled = acc[:, :32] / jnp.maximum(acc[:, 32:33], 1.0)
    logits = lax.dot_general(pooled, wl_ref[...], (((1,), (1,)), ((), ())),
                             preferred_element_type=jnp.float32)
    logits = logits + bl_ref[...]
    m = jnp.max(logits, axis=1, keepdims=True)
    e = jnp.exp(logits - m)
    out_ref[...] = logits - m - jnp.log(jnp.sum(e, axis=1, keepdims=True))


def _final(p, r3, b3, batch3, wl, bl):
  grid = N // BR
  return pl.pallas_call(
      _final_body,
      grid=(grid,),
      in_specs=[
          pl.BlockSpec((BR, 32), lambda i: (i, 0)),
          pl.BlockSpec((BR, 32), lambda i: (i + N // BR, 0)),
          pl.BlockSpec((BR, 32), lambda i: (i, 0)),
          pl.BlockSpec((1, 32), lambda i: (0, 0)),
          pl.BlockSpec((1, 1, BR), lambda i: (i, 0, 0)),
          pl.BlockSpec((C, 32), lambda i: (0, 0)),
          pl.BlockSpec((1, C), lambda i: (0, 0)),
      ],
      out_specs=pl.BlockSpec((G, C), lambda i: (0, 0)),
      out_shape=jax.ShapeDtypeStruct((G, C), jnp.float32),
      scratch_shapes=[pltpu.VMEM((G, 64), jnp.float32)],
  )(p, p, r3, b3.reshape(1, 32), batch3, wl, bl.reshape(1, C))


def kernel(x, edge_index, batch, W1_rel, b1, W1_root, g1, be1, W2_rel, b2,
           W2_root, g2, be2, W3_rel, b3, W3_root, Wl, bl):
  src = edge_index[0]
  dst = edge_index[1]
  pad = E_PAD - E
  srcm = jnp.concatenate([src, jnp.zeros((pad,), jnp.int32)]).reshape(
      NC * NS, CH, K)
  dstm = jnp.concatenate([dst, jnp.full((pad,), N, jnp.int32)]).reshape(
      NC * NS, CH, K)

  sc128 = _sc_segment_sum(128)
  sc64 = _sc_segment_sum(64, stage_y=True)
  sc32 = _sc_segment_sum(32, stage_y=True)
  z128 = jnp.zeros((K, 128), jnp.float32)
  z64 = jnp.zeros((K, 64), jnp.float32)
  z32 = jnp.zeros((K, 32), jnp.float32)

  y1, r1 = _mm(x, W1_rel, W1_root)
  p1 = sc128(y1, srcm, dstm, z128)
  zz1, s1 = _stats(p1, r1)
  y2, r2 = _bnmm(zz1, s1, g1, be1, W2_rel, W2_root)
  p2 = sc64(y2, srcm, dstm, z64)
  zz2, s2 = _stats(p2, r2)
  y3, r3 = _bnmm(zz2, s2, g2, be2, W3_rel, W3_root)
  p3 = sc32(y3, srcm, dstm, z32)
  batch3 = batch.reshape(N // BR, 1, BR)
  return _final(p3, r3, b3, batch3, Wl, bl)


# speedup vs baseline: 8.8071x; 1.3216x over previous
import functools

import jax
import jax.numpy as jnp
from jax import lax
from jax.experimental import pallas as pl
from jax.experimental.pallas import tpu as pltpu
from jax.experimental.pallas import tpu_sc as plsc

N = 10000
E = 160000
G = 64
C = 36

NC = 2
NS = 16
K = 128
CH = 40
E_PAD = NC * NS * CH * K
NPAD = 10240
ZCH = NPAD // NS // K
STRIPE = 624
TAIL = N - NS * STRIPE
OCH = [K, K, K, K, STRIPE - 4 * K]

BR = 2000


def _sc_segment_sum(F, stage_y=False, col_split=False):
  nch = 2 * CH if col_split else CH
  stage_y = stage_y or col_split
  mesh = plsc.VectorSubcoreMesh(core_axis_name="c", subcore_axis_name="s",
                                num_cores=NC, num_subcores=NS)
  scratch = [
      pltpu.VMEM((nch, K), jnp.int32),
      pltpu.VMEM((nch, K), jnp.int32),
      pltpu.VMEM((K, F), jnp.float32),
      pltpu.VMEM((K, F), jnp.float32),
      pltpu.VMEM_SHARED((NPAD, F), jnp.float32),
      pltpu.SemaphoreType.DMA,
      pltpu.SemaphoreType.DMA,
  ]
  if stage_y:
    scratch.append(pltpu.VMEM_SHARED((N, F), jnp.float32))

  @functools.partial(
      pl.kernel,
      out_type=jax.ShapeDtypeStruct((2 * N, F), jnp.float32),
      mesh=mesh,
      scratch_types=scratch,
      compiler_params=pltpu.CompilerParams(use_tc_tiling_on_sc=False),
  )
  def sc(y_hbm, srcm_hbm, dstm_hbm, zer_hbm, out_hbm,
         src_v, dst_v, rows0_v, rows1_v, acc_sh, sem0, sem1, *maybe_ysh):
    cid = lax.axis_index("c")
    sid = lax.axis_index("s")
    wid = sid if col_split else cid * NS + sid
    pltpu.sync_copy(srcm_hbm.at[wid], src_v)
    pltpu.sync_copy(dstm_hbm.at[wid], dst_v)
    if stage_y:
      ysh = maybe_ysh[0]
      ybase = cid * N if col_split else 0
      off = 0
      for w in OCH:
        r0 = sid * STRIPE + off
        pltpu.sync_copy(y_hbm.at[pl.ds(ybase + r0, w)], rows1_v.at[pl.ds(0, w)])
        pltpu.sync_copy(rows1_v.at[pl.ds(0, w)], ysh.at[pl.ds(r0, w)])
        off += w

      @pl.when(sid == NS - 1)
      def _():
        t0 = NS * STRIPE
        pltpu.sync_copy(y_hbm.at[pl.ds(ybase + t0, TAIL)],
                        rows1_v.at[pl.ds(0, TAIL)])
        pltpu.sync_copy(rows1_v.at[pl.ds(0, TAIL)], ysh.at[pl.ds(t0, TAIL)])

      ysrc = ysh
    else:
      ysrc = y_hbm
    pltpu.sync_copy(zer_hbm, rows0_v)
    for z in range(ZCH):
      pltpu.sync_copy(rows0_v, acc_sh.at[pl.ds(sid * (ZCH * K) + z * K, K)])
    plsc.subcore_barrier()

    pltpu.async_copy(ysrc.at[src_v.at[0]], rows0_v, sem0)

    def body(c2, carry):
      c = 2 * c2
      pltpu.make_async_copy(ysrc.at[src_v.at[c]], rows0_v, sem0).wait()
      pltpu.async_copy(ysrc.at[src_v.at[c + 1]], rows1_v, sem1)
      pltpu.sync_copy(rows0_v, acc_sh.at[dst_v.at[c]], add=True)
      pltpu.make_async_copy(ysrc.at[src_v.at[c + 1]], rows1_v, sem1).wait()

      @pl.when(c + 2 < nch)
      def _():
        pltpu.async_copy(ysrc.at[src_v.at[c + 2]], rows0_v, sem0)

      pltpu.sync_copy(rows1_v, acc_sh.at[dst_v.at[c + 1]], add=True)
      return carry

    lax.fori_loop(0, nch // 2, body, 0)
    plsc.subcore_barrier()
    off = 0
    for w in OCH:
      r0 = sid * STRIPE + off
      pltpu.sync_copy(acc_sh.at[pl.ds(r0, w)], rows0_v.at[pl.ds(0, w)])
      pltpu.sync_copy(rows0_v.at[pl.ds(0, w)],
                      out_hbm.at[pl.ds(cid * N + r0, w)])
      off += w

    @pl.when(sid == NS - 1)
    def _():
      t0 = NS * STRIPE
      pltpu.sync_copy(acc_sh.at[pl.ds(t0, TAIL)], rows0_v.at[pl.ds(0, TAIL)])
      pltpu.sync_copy(rows0_v.at[pl.ds(0, TAIL)],
                      out_hbm.at[pl.ds(cid * N + t0, TAIL)])

  return sc


def _mm_body(x_ref, wr_ref, wt_ref, y_ref, r_ref):
  xb = x_ref[...]
  dn = (((1,), (1,)), ((), ()))
  y_ref[...] = lax.dot_general(xb, wr_ref[...], dn,
                               preferred_element_type=jnp.float32)
  r_ref[...] = lax.dot_general(xb, wt_ref[...], dn,
                               preferred_element_type=jnp.float32)


def _mm(x, w_rel, w_root):
  fin = x.shape[1]
  fout = w_rel.shape[0]
  grid = N // BR
  return pl.pallas_call(
      _mm_body,
      grid=(grid,),
      in_specs=[
          pl.BlockSpec((BR, fin), lambda i: (i, 0)),
          pl.BlockSpec((fout, fin), lambda i: (0, 0)),
          pl.BlockSpec((fout, fin), lambda i: (0, 0)),
      ],
      out_specs=[
          pl.BlockSpec((BR, fout), lambda i: (i, 0)),
          pl.BlockSpec((BR, fout), lambda i: (i, 0)),
      ],
      out_shape=[
          jax.ShapeDtypeStruct((N, fout), jnp.float32),
          jax.ShapeDtypeStruct((N, fout), jnp.float32),
      ],
  )(x, w_rel, w_root)


def _mm_split_body(x_ref, wr_ref, wt_ref, y_ref, r_ref):
  xb = x_ref[...]
  dn = (((1,), (1,)), ((), ()))
  y_ref[...] = lax.dot_general(xb, wr_ref[0], dn,
                               preferred_element_type=jnp.float32)
  r_ref[...] = lax.dot_general(xb, wt_ref[0], dn,
                               preferred_element_type=jnp.float32)


def _mm_split(x, w_rel, w_root):
  fin = x.shape[1]
  fh = w_rel.shape[0] // 2
  grid = (2, N // BR)
  return pl.pallas_call(
      _mm_split_body,
      grid=grid,
      in_specs=[
          pl.BlockSpec((BR, fin), lambda h, i: (i, 0)),
          pl.BlockSpec((1, fh, fin), lambda h, i: (h, 0, 0)),
          pl.BlockSpec((1, fh, fin), lambda h, i: (h, 0, 0)),
      ],
      out_specs=[
          pl.BlockSpec((BR, fh), lambda h, i: (h * (N // BR) + i, 0)),
          pl.BlockSpec((BR, fh), lambda h, i: (h * (N // BR) + i, 0)),
      ],
      out_shape=[
          jax.ShapeDtypeStruct((2 * N, fh), jnp.float32),
          jax.ShapeDtypeStruct((2 * N, fh), jnp.float32),
      ],
  )(x, w_rel.reshape(2, fh, fin), w_root.reshape(2, fh, fin))


def _stats_cat_body(p0_ref, p1_ref, r0_ref, r1_ref, z_ref, s_ref):
  i = pl.program_id(0)
  z = jnp.concatenate(
      [p0_ref[...] + r0_ref[...], p1_ref[...] + r1_ref[...]], axis=1)
  z_ref[...] = z
  cs = jnp.sum(z, axis=0, keepdims=True)
  cq = jnp.sum(z * z, axis=0, keepdims=True)
  blk = jnp.concatenate(
      [cs, cq, jnp.zeros((6, z.shape[1]), jnp.float32)], axis=0)

  @pl.when(i == 0)
  def _():
    s_ref[...] = blk

  @pl.when(i != 0)
  def _():
    s_ref[...] = s_ref[...] + blk


def _stats_cat(p, r):
  fh = r.shape[1]
  grid = N // BR
  half = pl.BlockSpec((BR, fh), lambda i: (i, 0))
  half2 = pl.BlockSpec((BR, fh), lambda i: (i + N // BR, 0))
  return pl.pallas_call(
      _stats_cat_body,
      grid=(grid,),
      in_specs=[half, half2, half, half2],
      out_specs=[
          pl.BlockSpec((BR, 2 * fh), lambda i: (i, 0)),
          pl.BlockSpec((8, 2 * fh), lambda i: (0, 0)),
      ],
      out_shape=[
          jax.ShapeDtypeStruct((N, 2 * fh), jnp.float32),
          jax.ShapeDtypeStruct((8, 2 * fh), jnp.float32),
      ],
  )(p, p, r, r)


def _stats_body(p0_ref, p1_ref, r_ref, z_ref, s_ref):
  i = pl.program_id(0)
  z = p0_ref[...] + p1_ref[...] + r_ref[...]
  z_ref[...] = z
  cs = jnp.sum(z, axis=0, keepdims=True)
  cq = jnp.sum(z * z, axis=0, keepdims=True)
  blk = jnp.concatenate(
      [cs, cq, jnp.zeros((6, z.shape[1]), jnp.float32)], axis=0)

  @pl.when(i == 0)
  def _():
    s_ref[...] = blk

  @pl.when(i != 0)
  def _():
    s_ref[...] = s_ref[...] + blk


def _stats(p, r):
  f = r.shape[1]
  grid = N // BR
  return pl.pallas_call(
      _stats_body,
      grid=(grid,),
      in_specs=[
          pl.BlockSpec((BR, f), lambda i: (i, 0)),
          pl.BlockSpec((BR, f), lambda i: (i + N // BR, 0)),
          pl.BlockSpec((BR, f), lambda i: (i, 0)),
      ],
      out_specs=[
          pl.BlockSpec((BR, f), lambda i: (i, 0)),
          pl.BlockSpec((8, f), lambda i: (0, 0)),
      ],
      out_shape=[
          jax.ShapeDtypeStruct((N, f), jnp.float32),
          jax.ShapeDtypeStruct((8, f), jnp.float32),
      ],
  )(p, p, r)


def _bnmm_body(z_ref, s_ref, g_ref, be_ref, wr_ref, wt_ref, y_ref, r_ref):
  s = s_ref[...]
  mu = s[0:1, :] * (1.0 / N)
  var = s[1:2, :] * (1.0 / N) - mu * mu
  scale = g_ref[...] * lax.rsqrt(var + 1e-5)
  shift = be_ref[...] - mu * scale
  h = jnp.maximum(z_ref[...] * scale + shift, 0.0)
  dn = (((1,), (1,)), ((), ()))
  y_ref[...] = lax.dot_general(h, wr_ref[...], dn,
                               preferred_element_type=jnp.float32)
  r_ref[...] = lax.dot_general(h, wt_ref[...], dn,
                               preferred_element_type=jnp.float32)


def _bnmm(z, s, g, be, w_rel, w_root):
  fin = z.shape[1]
  fout = w_rel.shape[0]
  grid = N // BR
  return pl.pallas_call(
      _bnmm_body,
      grid=(grid,),
      in_specs=[
          pl.BlockSpec((BR, fin), lambda i: (i, 0)),
          pl.BlockSpec((8, fin), lambda i: (0, 0)),
          pl.BlockSpec((1, fin), lambda i: (0, 0)),
          pl.BlockSpec((1, fin), lambda i: (0, 0)),
          pl.BlockSpec((fout, fin), lambda i: (0, 0)),
          pl.BlockSpec((fout, fin), lambda i: (0, 0)),
      ],
      out_specs=[
          pl.BlockSpec((BR, fout), lambda i: (i, 0)),
          pl.BlockSpec((BR, fout), lambda i: (i, 0)),
      ],
      out_shape=[
          jax.ShapeDtypeStruct((N, fout), jnp.float32),
          jax.ShapeDtypeStruct((N, fout), jnp.float32),
      ],
  )(z, s, g.reshape(1, fin), be.reshape(1, fin), w_rel, w_root)


def _final_body(p0_ref, p1_ref, r_ref, b3_ref, bt_ref, wl_ref, bl_ref,
                out_ref, acc_ref):
  i = pl.program_id(0)

  @pl.when(i == 0)
  def _():
    acc_ref[...] = jnp.zeros_like(acc_ref)

  h = jnp.maximum(p0_ref[...] + p1_ref[...] + r_ref[...] + b3_ref[...], 0.0)
  bt = bt_ref[0, 0, :]
  oh = (lax.broadcasted_iota(jnp.int32, (G, BR), 0) == bt[None, :]).astype(
      jnp.float32)
  hcat = jnp.concatenate(
      [h, jnp.ones((BR, 1), jnp.float32), jnp.zeros((BR, 31), jnp.float32)],
      axis=1)
  acc_ref[...] += lax.dot_general(oh, hcat, (((1,), (0,)), ((), ())),
                                  preferred_element_type=jnp.float32)

  @pl.when(i == pl.num_programs(0) - 1)
  def _():
    acc = acc_ref[...]
    pooled = acc[:, :32] / jnp.maximum(acc[:, 32:33], 1.0)
    logits = lax.dot_general(pooled, wl_ref[...], (((1,), (1,)), ((), ())),
                             preferred_element_type=jnp.float32)
    logits = logits + bl_ref[...]
    m = jnp.max(logits, axis=1, keepdims=True)
    e = jnp.exp(logits - m)
    out_ref[...] = logits - m - jnp.log(jnp.sum(e, axis=1, keepdims=True))


def _final(p, r3, b3, batch3, wl, bl):
  grid = N // BR
  return pl.pallas_call(
      _final_body,
      grid=(grid,),
      in_specs=[
          pl.BlockSpec((BR, 32), lambda i: (i, 0)),
          pl.BlockSpec((BR, 32), lambda i: (i + N // BR, 0)),
          pl.BlockSpec((BR, 32), lambda i: (i, 0)),
          pl.BlockSpec((1, 32), lambda i: (0, 0)),
          pl.BlockSpec((1, 1, BR), lambda i: (i, 0, 0)),
          pl.BlockSpec((C, 32), lambda i: (0, 0)),
          pl.BlockSpec((1, C), lambda i: (0, 0)),
      ],
      out_specs=pl.BlockSpec((G, C), lambda i: (0, 0)),
      out_shape=jax.ShapeDtypeStruct((G, C), jnp.float32),
      scratch_shapes=[pltpu.VMEM((G, 64), jnp.float32)],
  )(p, p, r3, b3.reshape(1, 32), batch3, wl, bl.reshape(1, C))


def kernel(x, edge_index, batch, W1_rel, b1, W1_root, g1, be1, W2_rel, b2,
           W2_root, g2, be2, W3_rel, b3, W3_root, Wl, bl):
  src = edge_index[0]
  dst = edge_index[1]
  pad = E_PAD - E
  srcm = jnp.concatenate([src, jnp.zeros((pad,), jnp.int32)]).reshape(
      NC * NS, CH, K)
  dstm = jnp.concatenate([dst, jnp.full((pad,), N, jnp.int32)]).reshape(
      NC * NS, CH, K)

  srcm16 = srcm.reshape(NS, 2 * CH, K)
  dstm16 = dstm.reshape(NS, 2 * CH, K)

  sc64cs = _sc_segment_sum(64, col_split=True)
  sc64 = _sc_segment_sum(64, stage_y=True)
  sc32 = _sc_segment_sum(32, stage_y=True)
  z64 = jnp.zeros((K, 64), jnp.float32)
  z32 = jnp.zeros((K, 32), jnp.float32)

  ycat, rcat = _mm_split(x, W1_rel, W1_root)
  p1 = sc64cs(ycat, srcm16, dstm16, z64)
  zz1, s1 = _stats_cat(p1, rcat)
  y2, r2 = _bnmm(zz1, s1, g1, be1, W2_rel, W2_root)
  p2 = sc64(y2, srcm, dstm, z64)
  zz2, s2 = _stats(p2, r2)
  y3, r3 = _bnmm(zz2, s2, g2, be2, W3_rel, W3_root)
  p3 = sc32(y3, srcm, dstm, z32)
  batch3 = batch.reshape(N // BR, 1, BR)
  return _final(p3, r3, b3, batch3, Wl, bl)
